# baseline (speedup 1.0000x reference)
import functools
import jax
import jax.numpy as jnp
from jax import lax
from jax.experimental import pallas as pl
from jax.experimental.pallas import tpu as pltpu
from jax.experimental.pallas import tpu_sc as plsc

K = 80
EPS = 1e-5
B, N = 4, 2048
NB = 128
NBI = 256
NW = 32
RPW = B * N // NW
GRP = 16


def _bisect_body(pw_ref, t_ref, cg_ref, fl_ref):
    v = pw_ref[0, 0]
    i = jax.lax.bitcast_convert_type(v, jnp.int32)
    t = i ^ (jax.lax.shift_right_arithmetic(i, 31) & jnp.int32(0x7FFFFFFF))
    thi = jax.lax.shift_right_arithmetic(t, 16).astype(jnp.int16)
    tlo = (t ^ jnp.int32(0x8000)).astype(jnp.int16)

    def cnth(c):
        cb = c.astype(jnp.int16)[:, None]
        return jnp.sum((thi >= cb).astype(jnp.int16), axis=1).astype(jnp.int32)

    z32 = jnp.zeros((NBI,), jnp.int32)
    m32 = jnp.full((NBI,), jnp.int32(-0x8000))
    ph = jnp.where(cnth(z32) >= K, z32, m32)
    for j in range(14, -1, -1):
        cand = ph | jnp.int32(1 << j)
        ph = jnp.where(cnth(cand) >= K, cand, ph)
    ph16 = ph.astype(jnp.int16)[:, None]
    meq = thi == ph16
    cgt_hi = jnp.sum((thi > ph16).astype(jnp.int16), axis=1).astype(jnp.int32)

    def cntl(c):
        cb = c.astype(jnp.int16)[:, None]
        return cgt_hi + jnp.sum(
            (meq & (tlo >= cb)).astype(jnp.int16), axis=1).astype(jnp.int32)

    pb = jnp.where(cntl(z32) >= K, z32, m32)
    for j in range(14, -1, -1):
        cand = pb | jnp.int32(1 << j)
        pb = jnp.where(cntl(cand) >= K, cand, pb)
    p = (ph << 16) | ((pb & 0xFFFF) ^ 0x8000)
    tv = p ^ (jax.lax.shift_right_arithmetic(p, 31) & jnp.int32(0x7FFFFFFF))
    t_ref[0, 0, 0] = jax.lax.bitcast_convert_type(tv, jnp.float32)
    pb16 = pb.astype(jnp.int16)[:, None]
    cgt = cgt_hi + jnp.sum((meq & (tlo > pb16)).astype(jnp.int16),
                           axis=1).astype(jnp.int32)
    cge = cgt_hi + jnp.sum((meq & (tlo >= pb16)).astype(jnp.int16),
                           axis=1).astype(jnp.int32)
    cg_ref[0, 0, 0] = cgt
    fl_ref[0, 0, 0] = jnp.max((cge > K).astype(jnp.int32).reshape(NBI // 16, 16),
                              axis=1)


def _bisect(pw):
    nblk = N // NBI
    pw4 = pw.reshape(B, nblk, NBI, N)
    tv, cgt, flg = pl.pallas_call(
        _bisect_body,
        grid=(B, nblk),
        in_specs=[pl.BlockSpec((1, 1, NBI, N), lambda b, i: (b, i, 0, 0))],
        out_specs=[pl.BlockSpec((1, 1, 1, NBI), lambda b, i: (b, i, 0, 0)),
                   pl.BlockSpec((1, 1, 1, NBI), lambda b, i: (b, i, 0, 0)),
                   pl.BlockSpec((1, 1, 1, NBI // 16), lambda b, i: (b, i, 0, 0))],
        out_shape=[jax.ShapeDtypeStruct((B, nblk, 1, NBI), jnp.float32),
                   jax.ShapeDtypeStruct((B, nblk, 1, NBI), jnp.int32),
                   jax.ShapeDtypeStruct((B, nblk, 1, NBI // 16), jnp.int32)],
    )(pw4)
    return (tv.reshape(B * N), cgt.reshape(B * N),
            flg.reshape(B * N // 16))


def _sc_body(pw_hbm, tv_hbm, cgt_hbm, flg_hbm, xt_hbm, out_hbm,
             pwb0, pwb1, tvb, cgtb, flgs, idxb0, idxb1, gdst0, gdst1,
             sem_pw, sem_g0, sem_g1, sem_o0, sem_o1):
    wid = lax.axis_index("s") * 2 + lax.axis_index("c")
    rowbase = wid * RPW
    bof = (rowbase // N) * N
    pltpu.sync_copy(tv_hbm.at[pl.ds(rowbase, RPW)], tvb)
    pltpu.sync_copy(cgt_hbm.at[pl.ds(rowbase, RPW)], cgtb)
    pltpu.sync_copy(flg_hbm.at[pl.ds(wid * (RPW // GRP), RPW // GRP)], flgs)
    flvec = flgs[...]
    lanes = lax.iota(jnp.int32, 16)
    lanoff = lanes * N
    posbase = lanes * K
    zero = jnp.zeros((16,), jnp.int32)
    NG = RPW // GRP
    pwbufs = [pwb0, pwb1]
    idxbufs = [idxb0, idxb1]
    gdsts = [gdst0, gdst1]
    semg = [sem_g0, sem_g1]
    semo = [sem_o0, sem_o1]

    def start_pw(g):
        rows = rowbase + g * GRP
        return [pltpu.async_copy(pw_hbm.at[pl.ds(rows * N, GRP * N)],
                                 pwbufs[g % 2], sem_pw)]

    def scan(g):
        pwb = pwbufs[g % 2]
        idxb = idxbufs[g % 2]
        tvv = tvb[pl.ds(g * GRP, 16)]
        cgtv = cgtb[pl.ds(g * GRP, 16)]
        quota = K - cgtv

        def fast_col(j, run):
            colw = j + lanes
            col = jnp.where(colw >= N, colw - N, colw)
            v = plsc.load_gather(pwb, [lanoff + col])
            m = v >= tvv
            plsc.store_scatter(idxb, [posbase + run], bof + col, mask=m)
            return run + jnp.where(m, 1, 0)

        def tie_col(j, carry):
            rgt, req = carry
            v = plsc.load_gather(pwb, [lanoff + j])
            mge = v >= tvv
            mgt = v > tvv
            meq = mge & (~mgt)
            keep = mgt | (meq & (req < quota))
            pos = jnp.where(mgt, posbase + rgt, posbase + cgtv + req)
            plsc.store_scatter(idxb, [pos],
                               jnp.broadcast_to(bof + j, (16,)), mask=keep)
            return (rgt + jnp.where(mgt, 1, 0),
                    req + jnp.where(meq, 1, 0))

        def do_tie(_):
            lax.fori_loop(0, N, tie_col, (zero, zero), unroll=8)
            return 0

        def do_fast(_):
            lax.fori_loop(0, N, fast_col, zero, unroll=16)
            return 0

        lax.cond(flvec[g] > 0, do_tie, do_fast, 0)

    NC = GRP * K // 128

    def gather_out(g):
        idxb = idxbufs[g % 2]
        rows = rowbase + g * GRP
        cpg, cpo = {}, {}
        for c in range(NC):
            if c >= 2:
                cpo[c - 2].wait()
            cpg[c] = pltpu.async_copy(
                xt_hbm.at[idxb.at[pl.ds(c * 128, 128)]],
                gdsts[c % 2], semg[c % 2])
            if c >= 1:
                cpg[c - 1].wait()
                cpo[c - 1] = pltpu.async_copy(
                    gdsts[(c - 1) % 2],
                    out_hbm.at[pl.ds((rows * K) + (c - 1) * 128, 128)],
                    semo[(c - 1) % 2])
        cpg[NC - 1].wait()
        cpo[NC - 1] = pltpu.async_copy(
            gdsts[(NC - 1) % 2],
            out_hbm.at[pl.ds((rows * K) + (NC - 1) * 128, 128)],
            semo[(NC - 1) % 2])
        cpo[NC - 2].wait()
        cpo[NC - 1].wait()

    cp_pw = {0: start_pw(0)}
    for g in range(NG):
        for cp in cp_pw[g]:
            cp.wait()
        if g + 1 < NG:
            cp_pw[g + 1] = start_pw(g + 1)
        scan(g)
        gather_out(g)


def _sc_select_gather(pw, tv, cgt, flg, xt):
    D = xt.shape[1]
    mesh = plsc.VectorSubcoreMesh(core_axis_name="c", subcore_axis_name="s")
    f = pl.kernel(
        _sc_body,
        mesh=mesh,
        compiler_params=pltpu.CompilerParams(
            needs_layout_passes=False, use_tc_tiling_on_sc=False),
        out_type=jax.ShapeDtypeStruct((B * N * K, D), jnp.float32),
        scratch_types=[
            pltpu.VMEM((GRP * N,), jnp.float32),
            pltpu.VMEM((GRP * N,), jnp.float32),
            pltpu.VMEM((RPW,), jnp.float32),
            pltpu.VMEM((RPW,), jnp.int32),
            pltpu.VMEM((RPW // GRP,), jnp.int32),
            pltpu.VMEM((GRP * K,), jnp.int32),
            pltpu.VMEM((GRP * K,), jnp.int32),
            pltpu.VMEM((128, D), jnp.float32),
            pltpu.VMEM((128, D), jnp.float32),
            pltpu.SemaphoreType.DMA,
            pltpu.SemaphoreType.DMA,
            pltpu.SemaphoreType.DMA,
            pltpu.SemaphoreType.DMA,
            pltpu.SemaphoreType.DMA,
        ],
    )
    return f(pw, tv, cgt, flg, xt)


def _econv_body(xg_ref, xt_ref, w_ref, mx_ref, mn_ref, s_ref):
    xg = xg_ref[0]
    xc = xt_ref[0]
    D = xc.shape[1]
    O = w_ref.shape[0]
    diff = xg - xc[:, None, :]
    f = jnp.concatenate(
        [diff.astype(jnp.bfloat16),
         jnp.broadcast_to(xc[:, None, :], xg.shape).astype(jnp.bfloat16)],
        axis=2).reshape(NB * K, 2 * D)
    h = jax.lax.dot_general(f, w_ref[...].astype(jnp.bfloat16),
                            (((1,), (1,)), ((), ())),
                            preferred_element_type=jnp.float32)
    h = h.reshape(NB, K, O)
    mx_ref[0] = jnp.max(h, axis=1)
    mn_ref[0] = jnp.min(h, axis=1)
    O2 = O // 2
    ha, hb = h[:, :, :O2], h[:, :, O2:]
    s_ref[0, 0] = jnp.stack(
        [jnp.sum(ha), jnp.sum(hb), jnp.sum(ha * ha), jnp.sum(hb * hb),
         jnp.float32(0), jnp.float32(0), jnp.float32(0), jnp.float32(0)],
    )[None, :]


def _econv(xg, xt, W):
    Bb, Nn, Kk, D = xg.shape
    O = W.shape[0]
    nblk = Nn // NB
    return pl.pallas_call(
        _econv_body,
        grid=(Bb, nblk),
        in_specs=[pl.BlockSpec((1, NB, Kk, D), lambda b, i: (b, i, 0, 0)),
                  pl.BlockSpec((1, NB, D), lambda b, i: (b, i, 0)),
                  pl.BlockSpec((O, 2 * D), lambda b, i: (0, 0))],
        out_specs=[pl.BlockSpec((1, NB, O), lambda b, i: (b, i, 0)),
                   pl.BlockSpec((1, NB, O), lambda b, i: (b, i, 0)),
                   pl.BlockSpec((1, 1, 1, 8), lambda b, i: (b, i, 0, 0))],
        out_shape=[jax.ShapeDtypeStruct((Bb, Nn, O), jnp.float32),
                   jax.ShapeDtypeStruct((Bb, Nn, O), jnp.float32),
                   jax.ShapeDtypeStruct((Bb, nblk, 1, 8), jnp.float32)],
    )(xg, xt, W)


def _combine_body(mx_ref, mn_ref, st_ref, g_ref, b_ref, o_ref):
    mx, mn = mx_ref[0], mn_ref[0]
    st = st_ref[0]
    g, b = g_ref[0], b_ref[0]
    O = mx.shape[1]
    O2 = O // 2
    cnt = jnp.float32(O2 * N * K)
    S = jnp.sum(st[:, 0, :], axis=0)
    outs = []
    for gi in range(2):
        sl = slice(gi * O2, (gi + 1) * O2)
        mean = S[gi] / cnt
        var = S[2 + gi] / cnt - mean * mean
        scale = jax.lax.rsqrt(var + EPS)
        gg = g[sl][None, :]
        pick = jnp.where(gg >= 0, mx[:, sl], mn[:, sl])
        h = (pick - mean) * scale * gg + b[sl][None, :]
        outs.append(jnp.where(h >= 0, h, 0.2 * h))
    o_ref[0] = jnp.concatenate(outs, axis=1)


def _combine(mx, mn, st, g, b):
    Bb, Nn, O = mx.shape
    nblk = st.shape[1]
    spec = pl.BlockSpec((1, Nn, O), lambda bb: (bb, 0, 0))
    return pl.pallas_call(
        _combine_body,
        grid=(Bb,),
        in_specs=[spec, spec,
                  pl.BlockSpec((1, nblk, 1, 8), lambda bb: (bb, 0, 0, 0)),
                  pl.BlockSpec((1, O), lambda bb: (0, 0)),
                  pl.BlockSpec((1, O), lambda bb: (0, 0))],
        out_specs=spec,
        out_shape=jax.ShapeDtypeStruct((Bb, Nn, O), jnp.float32),
    )(mx, mn, st, g.reshape(1, O), b.reshape(1, O))


def _head_body(xf_ref, wm_ref, bm_ref, gm_ref, bb_ref, o_ref):
    xf = xf_ref[0]
    h = jax.lax.dot_general(xf, wm_ref[...], (((1,), (1,)), ((), ())),
                            preferred_element_type=jnp.float32)
    h = h + bm_ref[0][None, :]
    mx = jnp.max(h, axis=0)
    mn = jnp.min(h, axis=0)
    gm, bb = gm_ref[0], bb_ref[0]
    G, O2 = 8, 128
    cnt = jnp.float32(O2 * N)
    outs = []
    for gi in range(G):
        sl = slice(gi * O2, (gi + 1) * O2)
        hs = h[:, sl]
        mean = jnp.sum(hs) / cnt
        var = jnp.sum(hs * hs) / cnt - mean * mean
        scale = jax.lax.rsqrt(var + EPS)
        gg = gm[sl]
        pick = jnp.where(gg >= 0, mx[sl], mn[sl])
        hh = (pick - mean) * scale * gg + bb[sl]
        outs.append(jnp.maximum(hh, 0.0))
    o_ref[0, 0] = jnp.concatenate(outs)


def _head(xf, Wm, bm, gm, bmb):
    x4 = pl.pallas_call(
        _head_body,
        grid=(B,),
        in_specs=[pl.BlockSpec((1, N, 256), lambda bb: (bb, 0, 0)),
                  pl.BlockSpec((1024, 256), lambda bb: (0, 0)),
                  pl.BlockSpec((1, 1024), lambda bb: (0, 0)),
                  pl.BlockSpec((1, 1024), lambda bb: (0, 0)),
                  pl.BlockSpec((1, 1024), lambda bb: (0, 0))],
        out_specs=pl.BlockSpec((1, 1, 1024), lambda bb: (bb, 0, 0)),
        out_shape=jax.ShapeDtypeStruct((B, 1, 1024), jnp.float32),
    )(xf, Wm, bm.reshape(1, 1024), gm.reshape(1, 1024), bmb.reshape(1, 1024))
    return x4.reshape(B, 1024)


def _pw(xc):
    inner = -2.0 * jnp.einsum('bdn,bdm->bnm', xc, xc)
    xx = jnp.sum(xc * xc, axis=1)
    return -xx[:, :, None] - inner - xx[:, None, :]


def _layer(xt, xc, W, g, b):
    D = xt.shape[2]
    pw = _pw(xc)
    tv, cgt, flg = _bisect(pw)
    xg = _sc_select_gather(pw.reshape(B * N * N), tv, cgt, flg,
                           xt.reshape(B * N, D))
    xg = xg.reshape(B, N, K, D)
    mx, mn, st = _econv(xg, xt, W)
    return _combine(mx, mn, st, g, b)


@jax.jit
def kernel(x, W1, g1, b1, W2, g2, b2, W3, g3, b3, Wm, bm, gm, bmb):
    xt = jnp.transpose(x, (0, 2, 1))
    xt1 = jnp.pad(xt, ((0, 0), (0, 0), (0, 13)))
    W1p = jnp.concatenate(
        [jnp.pad(W1[:, :3], ((0, 0), (0, 13))),
         jnp.pad(W1[:, 3:], ((0, 0), (0, 13)))], axis=1)
    x1 = _layer(xt1, x, W1p, g1, b1)
    x2 = _layer(x1, jnp.transpose(x1, (0, 2, 1)), W2, g2, b2)
    x3 = _layer(x2, jnp.transpose(x2, (0, 2, 1)), W3, g3, b3)
    xf = jnp.concatenate([x1, x2, x3], axis=2)
    x4 = _head(xf, Wm, bm, gm, bmb)
    return x4, jnp.transpose(xf, (0, 2, 1))


# speedup vs baseline: 1.2610x; 1.2610x over previous
import functools
import jax
import jax.numpy as jnp
from jax import lax
from jax.experimental import pallas as pl
from jax.experimental.pallas import tpu as pltpu
from jax.experimental.pallas import tpu_sc as plsc

K = 80
EPS = 1e-5
B, N = 4, 2048
NB = 128
NBI = 256
NW = 32
RPW = B * N // NW
GRP = 16


def _bisect_body(pw_ref, t_ref, cg_ref, fl_ref):
    v = pw_ref[0, 0]
    i = jax.lax.bitcast_convert_type(v, jnp.int32)
    t = i ^ (jax.lax.shift_right_arithmetic(i, 31) & jnp.int32(0x7FFFFFFF))

    def cnt(th):
        return jnp.sum((t >= th[:, None]).astype(jnp.int32), axis=1)

    zero = jnp.zeros((NBI,), jnp.int32)
    minint = jnp.full((NBI,), jnp.int32(-0x80000000))
    p = jnp.where(cnt(zero) >= K, zero, minint)
    for j in range(30, -1, -1):
        cand = p | jnp.int32(1 << j)
        p = jnp.where(cnt(cand) >= K, cand, p)
    tv = p ^ (jax.lax.shift_right_arithmetic(p, 31) & jnp.int32(0x7FFFFFFF))
    t_ref[0, 0, 0] = jax.lax.bitcast_convert_type(tv, jnp.float32)
    cgt = jnp.sum((t > p[:, None]).astype(jnp.int32), axis=1)
    cge = jnp.sum((t >= p[:, None]).astype(jnp.int32), axis=1)
    cg_ref[0, 0, 0] = cgt
    fl_ref[0, 0, 0] = jnp.max((cge > K).astype(jnp.int32).reshape(NBI // 16, 16),
                              axis=1)


def _bisect(pw):
    nblk = N // NBI
    pw4 = pw.reshape(B, nblk, NBI, N)
    tv, cgt, flg = pl.pallas_call(
        _bisect_body,
        grid=(B, nblk),
        in_specs=[pl.BlockSpec((1, 1, NBI, N), lambda b, i: (b, i, 0, 0))],
        out_specs=[pl.BlockSpec((1, 1, 1, NBI), lambda b, i: (b, i, 0, 0)),
                   pl.BlockSpec((1, 1, 1, NBI), lambda b, i: (b, i, 0, 0)),
                   pl.BlockSpec((1, 1, 1, NBI // 16), lambda b, i: (b, i, 0, 0))],
        out_shape=[jax.ShapeDtypeStruct((B, nblk, 1, NBI), jnp.float32),
                   jax.ShapeDtypeStruct((B, nblk, 1, NBI), jnp.int32),
                   jax.ShapeDtypeStruct((B, nblk, 1, NBI // 16), jnp.int32)],
    )(pw4)
    return (tv.reshape(B * N), cgt.reshape(B * N),
            flg.reshape(B * N // 16))


def _sc_body(pw_hbm, tv_hbm, cgt_hbm, flg_hbm, xt_hbm, out_hbm,
             pwb0, pwb1, tvb, cgtb, flgs, idxb0, idxb1, gdst0, gdst1,
             sem_pw, sem_g0, sem_g1, sem_o0, sem_o1):
    wid = lax.axis_index("s") * 2 + lax.axis_index("c")
    rowbase = wid * RPW
    bof = (rowbase // N) * N
    pltpu.sync_copy(tv_hbm.at[pl.ds(rowbase, RPW)], tvb)
    pltpu.sync_copy(cgt_hbm.at[pl.ds(rowbase, RPW)], cgtb)
    pltpu.sync_copy(flg_hbm.at[pl.ds(wid * (RPW // GRP), RPW // GRP)], flgs)
    flvec = flgs[...]
    lanes = lax.iota(jnp.int32, 16)
    lanoff = lanes * N
    posbase = lanes * K
    zero = jnp.zeros((16,), jnp.int32)
    NG = RPW // GRP
    pwbufs = [pwb0, pwb1]
    idxbufs = [idxb0, idxb1]
    gdsts = [gdst0, gdst1]
    semg = [sem_g0, sem_g1]
    semo = [sem_o0, sem_o1]

    def start_pw(g):
        rows = rowbase + g * GRP
        return [pltpu.async_copy(pw_hbm.at[pl.ds(rows * N, GRP * N)],
                                 pwbufs[g % 2], sem_pw)]

    def scan(g):
        pwb = pwbufs[g % 2]
        idxb = idxbufs[g % 2]
        tvv = tvb[pl.ds(g * GRP, 16)]
        cgtv = cgtb[pl.ds(g * GRP, 16)]
        quota = K - cgtv

        def fast_col(j, run):
            colw = j + lanes
            col = jnp.where(colw >= N, colw - N, colw)
            v = plsc.load_gather(pwb, [lanoff + col])
            m = v >= tvv
            plsc.store_scatter(idxb, [posbase + run], bof + col, mask=m)
            return run + jnp.where(m, 1, 0)

        def tie_col(j, carry):
            rgt, req = carry
            v = plsc.load_gather(pwb, [lanoff + j])
            mge = v >= tvv
            mgt = v > tvv
            meq = mge & (~mgt)
            keep = mgt | (meq & (req < quota))
            pos = jnp.where(mgt, posbase + rgt, posbase + cgtv + req)
            plsc.store_scatter(idxb, [pos],
                               jnp.broadcast_to(bof + j, (16,)), mask=keep)
            return (rgt + jnp.where(mgt, 1, 0),
                    req + jnp.where(meq, 1, 0))

        def do_tie(_):
            lax.fori_loop(0, N, tie_col, (zero, zero), unroll=8)
            return 0

        def do_fast(_):
            lax.fori_loop(0, N, fast_col, zero, unroll=16)
            return 0

        lax.cond(flvec[g] > 0, do_tie, do_fast, 0)

    NC = GRP * K // 128

    def gather_out(g):
        idxb = idxbufs[g % 2]
        rows = rowbase + g * GRP
        cpg, cpo = {}, {}
        for c in range(NC):
            if c >= 2:
                cpo[c - 2].wait()
            cpg[c] = pltpu.async_copy(
                xt_hbm.at[idxb.at[pl.ds(c * 128, 128)]],
                gdsts[c % 2], semg[c % 2])
            if c >= 1:
                cpg[c - 1].wait()
                cpo[c - 1] = pltpu.async_copy(
                    gdsts[(c - 1) % 2],
                    out_hbm.at[pl.ds((rows * K) + (c - 1) * 128, 128)],
                    semo[(c - 1) % 2])
        cpg[NC - 1].wait()
        cpo[NC - 1] = pltpu.async_copy(
            gdsts[(NC - 1) % 2],
            out_hbm.at[pl.ds((rows * K) + (NC - 1) * 128, 128)],
            semo[(NC - 1) % 2])
        cpo[NC - 2].wait()
        cpo[NC - 1].wait()

    cp_pw = {0: start_pw(0)}
    for g in range(NG):
        for cp in cp_pw[g]:
            cp.wait()
        if g + 1 < NG:
            cp_pw[g + 1] = start_pw(g + 1)
        scan(g)
        gather_out(g)


def _sc_select_gather(pw, tv, cgt, flg, xt):
    D = xt.shape[1]
    mesh = plsc.VectorSubcoreMesh(core_axis_name="c", subcore_axis_name="s")
    f = pl.kernel(
        _sc_body,
        mesh=mesh,
        compiler_params=pltpu.CompilerParams(
            needs_layout_passes=False, use_tc_tiling_on_sc=False),
        out_type=jax.ShapeDtypeStruct((B * N * K, D), jnp.float32),
        scratch_types=[
            pltpu.VMEM((GRP * N,), jnp.float32),
            pltpu.VMEM((GRP * N,), jnp.float32),
            pltpu.VMEM((RPW,), jnp.float32),
            pltpu.VMEM((RPW,), jnp.int32),
            pltpu.VMEM((RPW // GRP,), jnp.int32),
            pltpu.VMEM((GRP * K,), jnp.int32),
            pltpu.VMEM((GRP * K,), jnp.int32),
            pltpu.VMEM((128, D), jnp.float32),
            pltpu.VMEM((128, D), jnp.float32),
            pltpu.SemaphoreType.DMA,
            pltpu.SemaphoreType.DMA,
            pltpu.SemaphoreType.DMA,
            pltpu.SemaphoreType.DMA,
            pltpu.SemaphoreType.DMA,
        ],
    )
    return f(pw, tv, cgt, flg, xt)


def _econv_body(xg_ref, xt_ref, w_ref, mx_ref, mn_ref, s_ref):
    xg = xg_ref[0]
    xc = xt_ref[0]
    D = xc.shape[1]
    O = w_ref.shape[0]
    diff = xg - xc[:, None, :]
    f = jnp.concatenate(
        [diff.astype(jnp.bfloat16),
         jnp.broadcast_to(xc[:, None, :], xg.shape).astype(jnp.bfloat16)],
        axis=2).reshape(NB * K, 2 * D)
    h = jax.lax.dot_general(f, w_ref[...].astype(jnp.bfloat16),
                            (((1,), (1,)), ((), ())),
                            preferred_element_type=jnp.float32)
    h = h.reshape(NB, K, O)
    mx_ref[0] = jnp.max(h, axis=1)
    mn_ref[0] = jnp.min(h, axis=1)
    O2 = O // 2
    ha, hb = h[:, :, :O2], h[:, :, O2:]
    s_ref[0, 0] = jnp.stack(
        [jnp.sum(ha), jnp.sum(hb), jnp.sum(ha * ha), jnp.sum(hb * hb),
         jnp.float32(0), jnp.float32(0), jnp.float32(0), jnp.float32(0)],
    )[None, :]


def _econv(xg, xt, W):
    Bb, Nn, Kk, D = xg.shape
    O = W.shape[0]
    nblk = Nn // NB
    return pl.pallas_call(
        _econv_body,
        grid=(Bb, nblk),
        in_specs=[pl.BlockSpec((1, NB, Kk, D), lambda b, i: (b, i, 0, 0)),
                  pl.BlockSpec((1, NB, D), lambda b, i: (b, i, 0)),
                  pl.BlockSpec((O, 2 * D), lambda b, i: (0, 0))],
        out_specs=[pl.BlockSpec((1, NB, O), lambda b, i: (b, i, 0)),
                   pl.BlockSpec((1, NB, O), lambda b, i: (b, i, 0)),
                   pl.BlockSpec((1, 1, 1, 8), lambda b, i: (b, i, 0, 0))],
        out_shape=[jax.ShapeDtypeStruct((Bb, Nn, O), jnp.float32),
                   jax.ShapeDtypeStruct((Bb, Nn, O), jnp.float32),
                   jax.ShapeDtypeStruct((Bb, nblk, 1, 8), jnp.float32)],
    )(xg, xt, W)


def _combine_body(mx_ref, mn_ref, st_ref, g_ref, b_ref, o_ref):
    mx, mn = mx_ref[0], mn_ref[0]
    st = st_ref[0]
    g, b = g_ref[0], b_ref[0]
    O = mx.shape[1]
    O2 = O // 2
    cnt = jnp.float32(O2 * N * K)
    S = jnp.sum(st[:, 0, :], axis=0)
    outs = []
    for gi in range(2):
        sl = slice(gi * O2, (gi + 1) * O2)
        mean = S[gi] / cnt
        var = S[2 + gi] / cnt - mean * mean
        scale = jax.lax.rsqrt(var + EPS)
        gg = g[sl][None, :]
        pick = jnp.where(gg >= 0, mx[:, sl], mn[:, sl])
        h = (pick - mean) * scale * gg + b[sl][None, :]
        outs.append(jnp.where(h >= 0, h, 0.2 * h))
    o_ref[0] = jnp.concatenate(outs, axis=1)


def _combine(mx, mn, st, g, b):
    Bb, Nn, O = mx.shape
    nblk = st.shape[1]
    spec = pl.BlockSpec((1, Nn, O), lambda bb: (bb, 0, 0))
    return pl.pallas_call(
        _combine_body,
        grid=(Bb,),
        in_specs=[spec, spec,
                  pl.BlockSpec((1, nblk, 1, 8), lambda bb: (bb, 0, 0, 0)),
                  pl.BlockSpec((1, O), lambda bb: (0, 0)),
                  pl.BlockSpec((1, O), lambda bb: (0, 0))],
        out_specs=spec,
        out_shape=jax.ShapeDtypeStruct((Bb, Nn, O), jnp.float32),
    )(mx, mn, st, g.reshape(1, O), b.reshape(1, O))


def _head_body(xf_ref, wm_ref, bm_ref, gm_ref, bb_ref, o_ref):
    xf = xf_ref[0]
    h = jax.lax.dot_general(xf, wm_ref[...], (((1,), (1,)), ((), ())),
                            preferred_element_type=jnp.float32)
    h = h + bm_ref[0][None, :]
    mx = jnp.max(h, axis=0)
    mn = jnp.min(h, axis=0)
    gm, bb = gm_ref[0], bb_ref[0]
    G, O2 = 8, 128
    cnt = jnp.float32(O2 * N)
    outs = []
    for gi in range(G):
        sl = slice(gi * O2, (gi + 1) * O2)
        hs = h[:, sl]
        mean = jnp.sum(hs) / cnt
        var = jnp.sum(hs * hs) / cnt - mean * mean
        scale = jax.lax.rsqrt(var + EPS)
        gg = gm[sl]
        pick = jnp.where(gg >= 0, mx[sl], mn[sl])
        hh = (pick - mean) * scale * gg + bb[sl]
        outs.append(jnp.maximum(hh, 0.0))
    o_ref[0, 0] = jnp.concatenate(outs)


def _head(xf, Wm, bm, gm, bmb):
    x4 = pl.pallas_call(
        _head_body,
        grid=(B,),
        in_specs=[pl.BlockSpec((1, N, 256), lambda bb: (bb, 0, 0)),
                  pl.BlockSpec((1024, 256), lambda bb: (0, 0)),
                  pl.BlockSpec((1, 1024), lambda bb: (0, 0)),
                  pl.BlockSpec((1, 1024), lambda bb: (0, 0)),
                  pl.BlockSpec((1, 1024), lambda bb: (0, 0))],
        out_specs=pl.BlockSpec((1, 1, 1024), lambda bb: (bb, 0, 0)),
        out_shape=jax.ShapeDtypeStruct((B, 1, 1024), jnp.float32),
    )(xf, Wm, bm.reshape(1, 1024), gm.reshape(1, 1024), bmb.reshape(1, 1024))
    return x4.reshape(B, 1024)


def _pw(xc):
    inner = -2.0 * jnp.einsum('bdn,bdm->bnm', xc, xc)
    xx = jnp.sum(xc * xc, axis=1)
    return -xx[:, :, None] - inner - xx[:, None, :]


def _layer(xt, xc, W, g, b):
    D = xt.shape[2]
    pw = _pw(xc)
    tv, cgt, flg = _bisect(pw)
    xg = _sc_select_gather(pw.reshape(B * N * N), tv, cgt, flg,
                           xt.reshape(B * N, D))
    xg = xg.reshape(B, N, K, D)
    mx, mn, st = _econv(xg, xt, W)
    return _combine(mx, mn, st, g, b)


@jax.jit
def kernel(x, W1, g1, b1, W2, g2, b2, W3, g3, b3, Wm, bm, gm, bmb):
    xt = jnp.transpose(x, (0, 2, 1))
    xt1 = jnp.pad(xt, ((0, 0), (0, 0), (0, 13)))
    W1p = jnp.concatenate(
        [jnp.pad(W1[:, :3], ((0, 0), (0, 13))),
         jnp.pad(W1[:, 3:], ((0, 0), (0, 13)))], axis=1)
    x1 = _layer(xt1, x, W1p, g1, b1)
    x2 = _layer(x1, jnp.transpose(x1, (0, 2, 1)), W2, g2, b2)
    x3 = _layer(x2, jnp.transpose(x2, (0, 2, 1)), W3, g3, b3)
    xf = jnp.concatenate([x1, x2, x3], axis=2)
    x4 = _head(xf, Wm, bm, gm, bmb)
    return x4, jnp.transpose(xf, (0, 2, 1))
